# baseline (speedup 1.0000x reference)
import functools
import jax
import jax.numpy as jnp
from jax import lax
from jax.experimental import pallas as pl
from jax.experimental.pallas import tpu as pltpu
from jax.experimental.pallas import tpu_sc as plsc

_NBUF = 2
_RPC = 4
_H = 2
_SP = 32
_G = 128


def _sc_gather_part(x1d, embedding, bh, s, base_row):
    info = plsc.get_sparse_core_info()
    nc = info.num_cores
    nw = nc * info.num_subcores
    d = embedding.shape[1]
    rpw = bh // nw
    n_chunks = rpw // _RPC
    cidx = _RPC * s
    ipw = rpw * s

    mesh = plsc.VectorSubcoreMesh(core_axis_name="c", subcore_axis_name="s")

    @functools.partial(
        pl.kernel,
        mesh=mesh,
        out_type=jax.ShapeDtypeStruct((bh, s, d), jnp.float32),
        scratch_types=[
            pltpu.VMEM((ipw,), jnp.int32),
            pltpu.VMEM((cidx, d), jnp.float32),
            pltpu.VMEM((cidx, d), jnp.float32),
            pltpu.SemaphoreType.DMA,
            pltpu.SemaphoreType.DMA,
            pltpu.SemaphoreType.DMA,
            pltpu.SemaphoreType.DMA,
        ],
    )
    def k(x_hbm, tab_hbm, out_hbm, idx_all, rows0, rows1, gs0, gs1, ss0, ss1):
        wid = lax.axis_index("s") * nc + lax.axis_index("c")
        base = wid * rpw
        rows = (rows0, rows1)
        gsem = (gs0, gs1)
        ssem = (ss0, ss1)

        pltpu.sync_copy(
            x_hbm.at[pl.ds((base_row + base) * s, ipw)], idx_all
        )

        def fire_gather(j, bf):
            pltpu.async_copy(
                tab_hbm.at[idx_all.at[pl.ds(j * cidx, cidx)]],
                rows[bf].at[pl.ds(0, cidx)],
                gsem[bf],
            )

        def wait_gather(j, bf):
            pltpu.make_async_copy(
                tab_hbm.at[idx_all.at[pl.ds(j * cidx, cidx)]],
                rows[bf].at[pl.ds(0, cidx)],
                gsem[bf],
            ).wait()

        def fire_scatters(j, bf):
            for i in range(_RPC):
                pltpu.async_copy(
                    rows[bf].at[pl.ds(i * s, s)],
                    out_hbm.at[base + j * _RPC + i],
                    ssem[bf],
                )

        def wait_scatters(j, bf):
            for i in range(_RPC):
                pltpu.make_async_copy(
                    rows[bf].at[pl.ds(i * s, s)],
                    out_hbm.at[base + j * _RPC + i],
                    ssem[bf],
                ).wait()

        for bf in range(_NBUF):
            fire_gather(bf, bf)

        def outer(i, carry):
            for bf in range(_NBUF):
                j = i * _NBUF + bf
                wait_gather(j, bf)
                fire_scatters(j, bf)
                wait_scatters(j, bf)
                fire_gather(j + _NBUF, bf)
            return carry

        lax.fori_loop(0, n_chunks // _NBUF - 1, outer, 0)

        for bf in range(_NBUF):
            j = n_chunks - _NBUF + bf
            wait_gather(j, bf)
            fire_scatters(j, bf)
        for bf in range(_NBUF):
            j = n_chunks - _NBUF + bf
            wait_scatters(j, bf)

    return k(x1d, embedding)


@functools.partial(jax.jit, static_argnames=("b", "s"))
def _embedding_lookup(x1d, embedding, b, s):
    bh = b // _H
    parts = [
        _sc_gather_part(x1d, embedding, bh, s, h * bh) for h in range(_H)
    ]
    acc = jnp.pad(parts[0], ((0, b - bh), (0, 0), (0, 0)))
    for h in range(1, _H):
        acc = lax.dynamic_update_slice(acc, parts[h], (h * bh, 0, 0))
    return acc


def kernel(X, embedding):
    b, s = X.shape
    info = plsc.get_sparse_core_info()
    nw = info.num_cores * info.num_subcores
    bh = b // _H
    rpw = bh // nw
    assert b == bh * _H and bh == nw * rpw
    assert rpw % (_RPC * _NBUF) == 0 and _RPC * s <= 128 and s <= _SP
    x1d = X.reshape(-1).astype(jnp.int32)
    return _embedding_lookup(x1d, embedding, b, s)


# speedup vs baseline: 1.5412x; 1.5412x over previous
import functools
import jax
import jax.numpy as jnp
from jax import lax
from jax.experimental import pallas as pl
from jax.experimental.pallas import tpu as pltpu
from jax.experimental.pallas import tpu_sc as plsc

_NBUF = 4
_RPC = 4


@functools.partial(jax.jit, static_argnames=("b", "s"))
def _sc_gather(x1d, embedding, b, s):
    info = plsc.get_sparse_core_info()
    nc = info.num_cores
    nw = nc * info.num_subcores
    d = embedding.shape[1]
    rpw = b // nw
    n_chunks = rpw // _RPC
    cidx = _RPC * s
    ipw = rpw * s

    mesh = plsc.VectorSubcoreMesh(core_axis_name="c", subcore_axis_name="s")

    @functools.partial(
        pl.kernel,
        mesh=mesh,
        out_type=jax.ShapeDtypeStruct((b, s, d), jnp.float32),
        scratch_types=[
            pltpu.VMEM((ipw,), jnp.int32),
        ]
        + [pltpu.VMEM((cidx, d), jnp.float32)] * _NBUF
        + [pltpu.SemaphoreType.DMA] * (2 * _NBUF),
    )
    def k(x_hbm, tab_hbm, out_hbm, idx_all, *bufs):
        rows = bufs[:_NBUF]
        gsem = bufs[_NBUF : 2 * _NBUF]
        ssem = bufs[2 * _NBUF :]
        wid = lax.axis_index("s") * nc + lax.axis_index("c")
        base = wid * rpw

        pltpu.sync_copy(x_hbm.at[pl.ds(wid * ipw, ipw)], idx_all)

        def fire_gather(j, bf):
            pltpu.async_copy(
                tab_hbm.at[idx_all.at[pl.ds(j * cidx, cidx)]], rows[bf], gsem[bf]
            )

        def wait_gather(j, bf):
            pltpu.make_async_copy(
                tab_hbm.at[idx_all.at[pl.ds(j * cidx, cidx)]], rows[bf], gsem[bf]
            ).wait()

        def fire_scatters(j, bf):
            for i in range(_RPC):
                pltpu.async_copy(
                    rows[bf].at[pl.ds(i * s, s)],
                    out_hbm.at[base + j * _RPC + i],
                    ssem[bf],
                )

        def wait_scatters(j, bf):
            for i in range(_RPC):
                pltpu.make_async_copy(
                    rows[bf].at[pl.ds(i * s, s)],
                    out_hbm.at[base + j * _RPC + i],
                    ssem[bf],
                ).wait()

        for bf in range(_NBUF):
            fire_gather(bf, bf)

        def outer(i, carry):
            for bf in range(_NBUF):
                j = i * _NBUF + bf
                wait_gather(j, bf)
                fire_scatters(j, bf)
                wait_scatters(j, bf)
                fire_gather(j + _NBUF, bf)
            return carry

        lax.fori_loop(0, n_chunks // _NBUF - 1, outer, 0)

        for bf in range(_NBUF):
            j = n_chunks - _NBUF + bf
            wait_gather(j, bf)
            fire_scatters(j, bf)
        for bf in range(_NBUF):
            j = n_chunks - _NBUF + bf
            wait_scatters(j, bf)

    return k(x1d, embedding)


def kernel(X, embedding):
    b, s = X.shape
    info = plsc.get_sparse_core_info()
    nw = info.num_cores * info.num_subcores
    rpw = b // nw
    assert b == nw * rpw and rpw % (_RPC * _NBUF) == 0 and _RPC * s <= 128
    x1d = X.reshape(-1).astype(jnp.int32)
    return _sc_gather(x1d, embedding, b, s)
